# Initial kernel scaffold; baseline (speedup 1.0000x reference)
import jax
import jax.numpy as jnp
from jax.experimental import pallas as pl


def kernel(true, logits):
    raise NotImplementedError("write your pallas kernel here")


# speedup vs baseline: 1.9000x; 1.9000x over previous
import functools

import jax
import jax.numpy as jnp
from jax import lax
from jax.experimental import pallas as pl
from jax.experimental.pallas import tpu as pltpu
from jax.experimental.pallas import tpu_sc as plsc

_NUM_CLASSES = 1000
_BATCH = 16384
_NC = 2
_NS = 16
_L = 16
_NW = _NC * _NS
_ROWS_PER_W = _BATCH // _NW
_TILE_ROWS = 32
_TILES = _ROWS_PER_W // _TILE_ROWS
_GROUPS = _TILE_ROWS // _L
_UNROLL = 8
_STEPS = _NUM_CLASSES // _UNROLL


def _tec_body(true_hbm, logits_hbm, out_hbm, buf, true_v, acc_v):
    wid = lax.axis_index("s") * _NC + lax.axis_index("c")
    row0 = wid * _ROWS_PER_W

    pltpu.sync_copy(true_hbm.at[pl.ds(row0, _ROWS_PER_W)], true_v)

    lane = lax.iota(jnp.int32, _L)

    def tile_body(t, acc):
        pltpu.sync_copy(
            logits_hbm.at[
                pl.ds((row0 + t * _TILE_ROWS) * _NUM_CLASSES,
                      _TILE_ROWS * _NUM_CLASSES)
            ],
            buf,
        )
        for g in range(_GROUPS):
            fbase = (lane + g * _L) * _NUM_CLASSES

            def col_body(_, carry):
                maxv, maxc, fidx = carry
                for _u in range(_UNROLL):
                    v = plsc.load_gather(buf, [fidx])
                    upd = v > maxv
                    maxv = jnp.where(upd, v, maxv)
                    maxc = jnp.where(upd, fidx, maxc)
                    fidx = fidx + 1
                return maxv, maxc, fidx

            init = (jnp.full((_L,), -jnp.inf, jnp.float32), fbase, fbase)
            _, maxc, _ = lax.fori_loop(0, _STEPS, col_body, init)

            true_vec = true_v[pl.ds(t * _TILE_ROWS + g * _L, _L)]
            acc = acc + (maxc == fbase + true_vec).astype(jnp.int32)
        return acc

    acc = lax.fori_loop(0, _TILES, tile_body, jnp.zeros((_L,), jnp.int32))
    acc_v[...] = acc
    pltpu.sync_copy(acc_v, out_hbm.at[wid])


_recall_counts = functools.partial(
    pl.kernel,
    out_type=jax.ShapeDtypeStruct((_NW, _L), jnp.int32),
    mesh=plsc.VectorSubcoreMesh(
        core_axis_name="c", subcore_axis_name="s",
        num_cores=_NC, num_subcores=_NS,
    ),
    scratch_types=[
        pltpu.VMEM((_TILE_ROWS * _NUM_CLASSES,), jnp.float32),
        pltpu.VMEM((_ROWS_PER_W,), jnp.int32),
        pltpu.VMEM((_L,), jnp.int32),
    ],
    compiler_params=pltpu.CompilerParams(needs_layout_passes=False),
)(_tec_body)


def kernel(true, logits):
    counts = _recall_counts(true, logits.reshape(_BATCH * _NUM_CLASSES))
    return counts.astype(jnp.float32).sum() / jnp.float32(_BATCH)


# speedup vs baseline: 2.0709x; 1.0900x over previous
import functools

import jax
import jax.numpy as jnp
from jax import lax
from jax.experimental import pallas as pl
from jax.experimental.pallas import tpu as pltpu
from jax.experimental.pallas import tpu_sc as plsc

_NUM_CLASSES = 1000
_BATCH = 16384
_NC = 2
_NS = 16
_L = 16
_NW = _NC * _NS
_ROWS_PER_W = _BATCH // _NW
_TILE_ROWS = 32
_TILES = _ROWS_PER_W // _TILE_ROWS
_GROUPS = _TILE_ROWS // _L
_UNROLL = 8
_STEPS = _NUM_CLASSES // _UNROLL


def _tec_body(true_hbm, logits_hbm, out_hbm, buf, true_v, acc_v):
    wid = lax.axis_index("s") * _NC + lax.axis_index("c")
    row0 = wid * _ROWS_PER_W

    pltpu.sync_copy(true_hbm.at[pl.ds(row0, _ROWS_PER_W)], true_v)

    lane = lax.iota(jnp.int32, _L)

    def tile_body(t, acc):
        pltpu.sync_copy(
            logits_hbm.at[
                pl.ds((row0 + t * _TILE_ROWS) * _NUM_CLASSES,
                      _TILE_ROWS * _NUM_CLASSES)
            ],
            buf,
        )
        for g in range(_GROUPS):
            fbase = (lane + g * _L) * _NUM_CLASSES

            def col_body(_, carry):
                maxv, maxc, base = carry
                maxv, maxc = list(maxv), list(maxc)
                for u in range(_UNROLL):
                    fidx = base + u
                    v = plsc.load_gather(buf, [fidx])
                    upd = v > maxv[u]
                    maxv[u] = jnp.where(upd, v, maxv[u])
                    maxc[u] = jnp.where(upd, fidx, maxc[u])
                return tuple(maxv), tuple(maxc), base + _UNROLL

            init = (
                tuple(jnp.full((_L,), -jnp.inf, jnp.float32)
                      for _ in range(_UNROLL)),
                tuple(fbase + u for u in range(_UNROLL)),
                fbase,
            )
            maxv, maxc, _ = lax.fori_loop(0, _STEPS, col_body, init)

            av, ac = maxv[0], maxc[0]
            for u in range(1, _UNROLL):
                better = (maxv[u] > av) | ((maxv[u] == av) & (maxc[u] < ac))
                av = jnp.where(better, maxv[u], av)
                ac = jnp.where(better, maxc[u], ac)

            true_vec = true_v[pl.ds(t * _TILE_ROWS + g * _L, _L)]
            acc = acc + (ac == fbase + true_vec).astype(jnp.int32)
        return acc

    acc = lax.fori_loop(0, _TILES, tile_body, jnp.zeros((_L,), jnp.int32))
    acc_v[...] = acc
    pltpu.sync_copy(acc_v, out_hbm.at[wid])


_recall_counts = functools.partial(
    pl.kernel,
    out_type=jax.ShapeDtypeStruct((_NW, _L), jnp.int32),
    mesh=plsc.VectorSubcoreMesh(
        core_axis_name="c", subcore_axis_name="s",
        num_cores=_NC, num_subcores=_NS,
    ),
    scratch_types=[
        pltpu.VMEM((_TILE_ROWS * _NUM_CLASSES,), jnp.float32),
        pltpu.VMEM((_ROWS_PER_W,), jnp.int32),
        pltpu.VMEM((_L,), jnp.int32),
    ],
    compiler_params=pltpu.CompilerParams(needs_layout_passes=False),
)(_tec_body)


def kernel(true, logits):
    counts = _recall_counts(true, logits.reshape(_BATCH * _NUM_CLASSES))
    return counts.astype(jnp.float32).sum() / jnp.float32(_BATCH)


# speedup vs baseline: 2.0715x; 1.0003x over previous
import functools

import jax
import jax.numpy as jnp
from jax import lax
from jax.experimental import pallas as pl
from jax.experimental.pallas import tpu as pltpu
from jax.experimental.pallas import tpu_sc as plsc

_NUM_CLASSES = 1000
_BATCH = 16384
_NC = 2
_NS = 16
_L = 16
_NW = _NC * _NS
_ROWS_PER_W = _BATCH // _NW
_TILE_ROWS = 32
_TILES = _ROWS_PER_W // _TILE_ROWS
_GROUPS = _TILE_ROWS // _L
_UNROLL = 8
_STEPS = _NUM_CLASSES // _UNROLL


def _tec_body(true_hbm, logits_hbm, out_hbm, buf, true_v, acc_v, dma_sem):
    wid = lax.axis_index("s") * _NC + lax.axis_index("c")
    row0 = wid * _ROWS_PER_W

    pltpu.sync_copy(true_hbm.at[pl.ds(row0, _ROWS_PER_W)], true_v)

    lane = lax.iota(jnp.int32, _L)

    def tile_body(t, acc):
        pltpu.async_copy(
            logits_hbm.at[
                pl.ds((row0 + t * _TILE_ROWS) * _NUM_CLASSES,
                      _TILE_ROWS * _NUM_CLASSES)
            ],
            buf,
            dma_sem,
        ).wait()
        for g in range(_GROUPS):
            fbase = (lane + g * _L) * _NUM_CLASSES

            def col_body(_, carry):
                maxv, maxc, base = carry
                maxv, maxc = list(maxv), list(maxc)
                for u in range(_UNROLL):
                    fidx = base + u
                    v = plsc.load_gather(buf, [fidx])
                    upd = v > maxv[u]
                    maxv[u] = jnp.where(upd, v, maxv[u])
                    maxc[u] = jnp.where(upd, fidx, maxc[u])
                return tuple(maxv), tuple(maxc), base + _UNROLL

            init = (
                tuple(jnp.full((_L,), -jnp.inf, jnp.float32)
                      for _ in range(_UNROLL)),
                tuple(fbase + u for u in range(_UNROLL)),
                fbase,
            )
            maxv, maxc, _ = lax.fori_loop(0, _STEPS, col_body, init)

            av, ac = maxv[0], maxc[0]
            for u in range(1, _UNROLL):
                better = (maxv[u] > av) | ((maxv[u] == av) & (maxc[u] < ac))
                av = jnp.where(better, maxv[u], av)
                ac = jnp.where(better, maxc[u], ac)

            true_vec = true_v[pl.ds(t * _TILE_ROWS + g * _L, _L)]
            acc = acc + (ac == fbase + true_vec).astype(jnp.int32)
        return acc

    acc = lax.fori_loop(0, _TILES, tile_body, jnp.zeros((_L,), jnp.int32))
    acc_v[...] = acc
    pltpu.sync_copy(acc_v, out_hbm.at[wid])


_recall_counts = functools.partial(
    pl.kernel,
    out_type=jax.ShapeDtypeStruct((_NW, _L), jnp.int32),
    mesh=plsc.VectorSubcoreMesh(
        core_axis_name="c", subcore_axis_name="s",
        num_cores=_NC, num_subcores=_NS,
    ),
    scratch_types=[
        pltpu.VMEM((_TILE_ROWS * _NUM_CLASSES,), jnp.float32),
        pltpu.VMEM((_ROWS_PER_W,), jnp.int32),
        pltpu.VMEM((_L,), jnp.int32),
        pltpu.SemaphoreType.DMA,
    ],
    compiler_params=pltpu.CompilerParams(needs_layout_passes=False),
)(_tec_body)


def kernel(true, logits):
    counts = _recall_counts(true, logits.reshape(_BATCH * _NUM_CLASSES))
    return counts.astype(jnp.float32).sum() / jnp.float32(_BATCH)
